# baseline (speedup 1.0000x reference)
import functools

import jax
import jax.numpy as jnp
from jax.experimental import pallas as pl
from jax.experimental.pallas import tpu as pltpu

N = 10000
E = 320000
B = 16
K = 5
NC = 40

_EPS = 1e-5


def _mlp_stats_body(x_ref, wt_ref, b_ref, y_ref, stats_ref):
    i = pl.program_id(0)
    x = x_ref[...]
    y = jnp.dot(x, wt_ref[...], preferred_element_type=jnp.float32)
    y = jnp.maximum(y + b_ref[0, :][None, :], 0.0)
    y_ref[...] = y
    s1 = jnp.sum(y, axis=0)
    s2 = jnp.sum(y * y, axis=0)

    @pl.when(i == 0)
    def _init():
        stats_ref[...] = jnp.zeros_like(stats_ref)

    upd = jnp.concatenate(
        [s1[None, :], s2[None, :], jnp.zeros((6, s1.shape[0]), jnp.float32)],
        axis=0)
    stats_ref[...] = stats_ref[...] + upd


def _mlp_pass(x, Wt, b, tile):
    M, IN = x.shape
    OUT = Wt.shape[1]
    grid = (M // tile,)
    y, stats = pl.pallas_call(
        _mlp_stats_body,
        grid=grid,
        in_specs=[
            pl.BlockSpec((tile, IN), lambda i: (i, 0)),
            pl.BlockSpec((IN, OUT), lambda i: (0, 0)),
            pl.BlockSpec((1, OUT), lambda i: (0, 0)),
        ],
        out_specs=[
            pl.BlockSpec((tile, OUT), lambda i: (i, 0)),
            pl.BlockSpec((8, OUT), lambda i: (0, 0)),
        ],
        out_shape=[
            jax.ShapeDtypeStruct((M, OUT), jnp.float32),
            jax.ShapeDtypeStruct((8, OUT), jnp.float32),
        ],
    )(x, Wt, b.reshape(1, OUT))
    return y, stats


def _make_knn(n, tr, tcol, kpad, k, interpret=False):
    nt = n // tr

    def body(tl_ref, th_ref, xrow_ref, xsT_ref, sq_ref, rs_ref, re_ref,
             out_ref):
        i = pl.program_id(0)
        lo = tl_ref[i]
        hi = th_ref[i]
        c0base = (lo // tcol) * tcol
        nsteps = (hi - c0base + tcol - 1) // tcol
        xrow = xrow_ref[...]
        rs = rs_ref[...]
        re = re_ref[...]
        neg = jnp.float32(-jnp.inf)
        bvals0 = jnp.full((tr, 128), neg, jnp.float32)
        bidx0 = jnp.zeros((tr, 128), jnp.int32)
        width = 128 + tcol

        def step(t, carry):
            bvals, bidx = carry
            c0 = c0base + t * tcol
            colx = xsT_ref[:, pl.ds(c0, tcol)]
            sqj = sq_ref[:, pl.ds(c0, tcol)]
            dots = jnp.dot(xrow, colx, preferred_element_type=jnp.float32)
            key = 2.0 * dots - sqj
            j = c0 + jax.lax.broadcasted_iota(jnp.int32, (tr, tcol), 1)
            mask = (j >= rs) & (j < re)
            key = jnp.where(mask, key, neg)
            cat_v = jnp.concatenate([bvals, key], axis=1)
            cat_i = jnp.concatenate([bidx, j], axis=1)
            pos = jax.lax.broadcasted_iota(jnp.int32, (tr, width), 1)
            nv, ni = [], []
            for _ in range(k):
                v = jnp.max(cat_v, axis=1, keepdims=True)
                elig = cat_v == v
                p = jnp.min(jnp.where(elig, pos, width), axis=1,
                            keepdims=True)
                sel = pos == p
                ik = jnp.sum(jnp.where(sel, cat_i, 0), axis=1, keepdims=True)
                nv.append(v)
                ni.append(ik)
                cat_v = jnp.where(sel, neg, cat_v)
            bvals = jnp.concatenate(
                nv + [jnp.full((tr, 128 - k), neg, jnp.float32)], axis=1)
            bidx = jnp.concatenate(
                ni + [jnp.zeros((tr, 128 - k), jnp.int32)], axis=1)
            return bvals, bidx

        _, bidx = jax.lax.fori_loop(0, nsteps, step, (bvals0, bidx0))
        seg = re - rs
        kk = jax.lax.broadcasted_iota(jnp.int32, (tr, kpad), 1)
        cand = kk - seg
        fill = jnp.where(cand < rs, cand, cand + seg)
        out_ref[...] = jnp.where(kk >= seg, fill, bidx[:, :kpad])

    npad = ((n + tcol - 1) // tcol) * tcol

    def knn(xs, row_start, row_end, tile_lo, tile_hi):
        xsT = jnp.pad(xs.T, ((0, 0), (0, npad - n)))
        sq2 = jnp.pad(jnp.sum(xs * xs, axis=1)[None, :], ((0, 0), (0, npad - n)))
        grid_spec = pltpu.PrefetchScalarGridSpec(
            num_scalar_prefetch=2,
            grid=(nt,),
            in_specs=[
                pl.BlockSpec((tr, xs.shape[1]), lambda i, *_: (i, 0)),
                pl.BlockSpec((xs.shape[1], npad), lambda i, *_: (0, 0)),
                pl.BlockSpec((1, npad), lambda i, *_: (0, 0)),
                pl.BlockSpec((tr, 1), lambda i, *_: (i, 0)),
                pl.BlockSpec((tr, 1), lambda i, *_: (i, 0)),
            ],
            out_specs=pl.BlockSpec((tr, kpad), lambda i, *_: (i, 0)),
        )
        return pl.pallas_call(
            body,
            grid_spec=grid_spec,
            interpret=interpret,
            out_shape=jax.ShapeDtypeStruct((n, kpad), jnp.int32),
        )(tile_lo, tile_hi, xs, xsT, sq2,
          row_start[:, None], row_end[:, None])

    return knn


def _lin1_pool_body(x1_ref, x2_ref, wa_ref, wb_ref, b_ref, batch_ref,
                    stats_ref, gmax_ref, gmin_ref, *, nb):
    i = pl.program_id(0)
    y = jnp.dot(x1_ref[...], wa_ref[...], preferred_element_type=jnp.float32)
    y = y + jnp.dot(x2_ref[...], wb_ref[...],
                    preferred_element_type=jnp.float32)
    y = jnp.maximum(y + b_ref[0, :][None, :], 0.0)
    s1 = jnp.sum(y, axis=0)
    s2 = jnp.sum(y * y, axis=0)
    zero_pad = jnp.zeros((6, y.shape[1]), jnp.float32)

    @pl.when(i == 0)
    def _init():
        stats_ref[...] = jnp.zeros_like(stats_ref)
        gmax_ref[...] = jnp.full_like(gmax_ref, -jnp.inf)
        gmin_ref[...] = jnp.full_like(gmin_ref, jnp.inf)

    stats_ref[...] = stats_ref[...] + jnp.concatenate(
        [s1[None, :], s2[None, :], zero_pad], axis=0)
    bvec = batch_ref[...]
    rows_max = []
    rows_min = []
    for g in range(nb):
        m = bvec == g
        rows_max.append(jnp.max(jnp.where(m, y, -jnp.inf), axis=0))
        rows_min.append(jnp.min(jnp.where(m, y, jnp.inf), axis=0))
    gmax_ref[...] = jnp.maximum(gmax_ref[...], jnp.stack(rows_max, axis=0))
    gmin_ref[...] = jnp.minimum(gmin_ref[...], jnp.stack(rows_min, axis=0))


def _lin1_pool(x1, x2, Wa_t, Wb_t, b, batch2d, tile, nb):
    n = x1.shape[0]
    f = Wa_t.shape[1]
    stats, gmax, gmin = pl.pallas_call(
        functools.partial(_lin1_pool_body, nb=nb),
        grid=(n // tile,),
        in_specs=[
            pl.BlockSpec((tile, x1.shape[1]), lambda i: (i, 0)),
            pl.BlockSpec((tile, x2.shape[1]), lambda i: (i, 0)),
            pl.BlockSpec((x1.shape[1], f), lambda i: (0, 0)),
            pl.BlockSpec((x2.shape[1], f), lambda i: (0, 0)),
            pl.BlockSpec((1, f), lambda i: (0, 0)),
            pl.BlockSpec((tile, 1), lambda i: (i, 0)),
        ],
        out_specs=[
            pl.BlockSpec((8, f), lambda i: (0, 0)),
            pl.BlockSpec((nb, f), lambda i: (0, 0)),
            pl.BlockSpec((nb, f), lambda i: (0, 0)),
        ],
        out_shape=[
            jax.ShapeDtypeStruct((8, f), jnp.float32),
            jax.ShapeDtypeStruct((nb, f), jnp.float32),
            jax.ShapeDtypeStruct((nb, f), jnp.float32),
        ],
    )(x1, x2, Wa_t, Wb_t, b.reshape(1, f), batch2d)
    return stats, gmax, gmin


def _bn_affine(stats, count, g, be):
    mean = stats[0] / count
    var = stats[1] / count - mean * mean
    inv = g / jnp.sqrt(var + _EPS)
    return inv, be - mean * inv


def kernel(pos, edge_index, batch, c1_W0, c1_b0, c1_g0, c1_be0, c1_W1, c1_b1,
           c1_g1, c1_be1, c1_W2, c1_b2, c1_g2, c1_be2, c2_W0, c2_b0, c2_g0,
           c2_be0, l1_W, l1_b, l1_g, l1_be, m_W0, m_b0, m_g0, m_be0, m_W1,
           m_b1, m_g1, m_be1, m_W2, m_b2):
    src = edge_index[0]
    dst = edge_index[1]

    xi = pos[dst]
    xj = pos[src]
    m0 = jnp.concatenate([xi, xj - xi, jnp.zeros((E, 2), jnp.float32)], axis=1)
    W0t = jnp.concatenate([c1_W0.T, jnp.zeros((2, 64), jnp.float32)], axis=0)

    y1, st1 = _mlp_pass(m0, W0t, c1_b0, tile=4000)
    a1, c1 = _bn_affine(st1, float(E), c1_g0, c1_be0)
    W1t = (c1_W1 * a1[None, :]).T
    b1f = c1_W1 @ c1 + c1_b1
    y2, st2 = _mlp_pass(y1, W1t, b1f, tile=4000)
    a2, c2 = _bn_affine(st2, float(E), c1_g1, c1_be1)
    W2t = (c1_W2 * a2[None, :]).T
    b2f = c1_W2 @ c2 + c1_b2
    y3, st3 = _mlp_pass(y2, W2t, b2f, tile=4000)
    a3, c3 = _bn_affine(st3, float(E), c1_g2, c1_be2)

    ymax = jax.ops.segment_max(y3, dst, num_segments=N)
    empty = jnp.isneginf(ymax[:, :1])
    x1 = jnp.where(empty, 0.0, a3[None, :] * ymax + c3[None, :])

    TR, TCOL = 400, 512
    bs = jnp.arange(B, dtype=batch.dtype)
    seg_lo = jnp.searchsorted(batch, bs, side='left').astype(jnp.int32)
    seg_hi = jnp.searchsorted(batch, bs, side='right').astype(jnp.int32)
    row_start = seg_lo[batch]
    row_end = seg_hi[batch]
    tile_lo = row_start[::TR]
    tile_hi = row_end[TR - 1::TR]
    idx = _make_knn(N, TR, TCOL, 8, K)(x1, row_start, row_end,
                                       tile_lo, tile_hi)[:, :K]
    xj2 = x1[idx]
    xi2 = jnp.broadcast_to(x1[:, None, :], (N, K, 64))
    m2 = jnp.concatenate([xi2, xj2 - xi2], axis=-1).reshape(N * K, 128)
    y4, st4 = _mlp_pass(m2, c2_W0.T, c2_b0, tile=4000)
    a4, c4 = _bn_affine(st4, float(N * K), c2_g0, c2_be0)
    z4 = a4[None, :] * y4 + c4[None, :]
    x2 = jnp.max(z4.reshape(N, K, 128), axis=1)

    Wa_t = l1_W[:, :64].T
    Wb_t = l1_W[:, 64:].T
    st5, gmax, gmin = _lin1_pool(x1, x2, Wa_t, Wb_t, l1_b,
                                 batch[:, None].astype(jnp.int32),
                                 tile=1000, nb=B)
    a5, c5 = _bn_affine(st5, float(N), l1_g, l1_be)
    g = jnp.where(a5[None, :] >= 0, a5[None, :] * gmax,
                  a5[None, :] * gmin) + c5[None, :]

    def head_layer(x, W, b, gg, bb):
        y = jax.nn.relu(x @ W.T + b)
        mean = jnp.mean(y, axis=0)
        var = jnp.var(y, axis=0)
        return gg * (y - mean) / jnp.sqrt(var + _EPS) + bb

    g = head_layer(g, m_W0, m_b0, m_g0, m_be0)
    g = head_layer(g, m_W1, m_b1, m_g1, m_be1)
    return g @ m_W2.T + m_b2


# speedup vs baseline: 1.0057x; 1.0057x over previous
import functools

import jax
import jax.numpy as jnp
from jax.experimental import pallas as pl
from jax.experimental.pallas import tpu as pltpu

N = 10000
E = 320000
B = 16
K = 5
NC = 40

_EPS = 1e-5


def _mlp_stats_body(x_ref, wt_ref, b_ref, y_ref, stats_ref):
    i = pl.program_id(0)
    x = x_ref[...]
    y = jnp.dot(x, wt_ref[...], preferred_element_type=jnp.float32)
    y = jnp.maximum(y + b_ref[0, :][None, :], 0.0)
    y_ref[...] = y
    s1 = jnp.sum(y, axis=0)
    s2 = jnp.sum(y * y, axis=0)

    @pl.when(i == 0)
    def _init():
        stats_ref[...] = jnp.zeros_like(stats_ref)

    upd = jnp.concatenate(
        [s1[None, :], s2[None, :], jnp.zeros((6, s1.shape[0]), jnp.float32)],
        axis=0)
    stats_ref[...] = stats_ref[...] + upd


def _mlp_pass(x, Wt, b, tile):
    M, IN = x.shape
    OUT = Wt.shape[1]
    grid = (M // tile,)
    y, stats = pl.pallas_call(
        _mlp_stats_body,
        grid=grid,
        in_specs=[
            pl.BlockSpec((tile, IN), lambda i: (i, 0)),
            pl.BlockSpec((IN, OUT), lambda i: (0, 0)),
            pl.BlockSpec((1, OUT), lambda i: (0, 0)),
        ],
        out_specs=[
            pl.BlockSpec((tile, OUT), lambda i: (i, 0)),
            pl.BlockSpec((8, OUT), lambda i: (0, 0)),
        ],
        out_shape=[
            jax.ShapeDtypeStruct((M, OUT), jnp.float32),
            jax.ShapeDtypeStruct((8, OUT), jnp.float32),
        ],
    )(x, Wt, b.reshape(1, OUT))
    return y, stats


def _make_knn(n, tr, tcol, kpad, k, interpret=False):
    nt = n // tr

    def body(tl_ref, th_ref, xrow_ref, xsT_ref, sq_ref, rs_ref, re_ref,
             out_ref):
        i = pl.program_id(0)
        lo = tl_ref[i]
        hi = th_ref[i]
        c0base = (lo // tcol) * tcol
        nsteps = (hi - c0base + tcol - 1) // tcol
        xrow = xrow_ref[...]
        rs = rs_ref[...]
        re = re_ref[...]
        neg = jnp.float32(-jnp.inf)
        bvals0 = jnp.full((tr, 128), neg, jnp.float32)
        bidx0 = jnp.zeros((tr, 128), jnp.int32)
        width = 128 + tcol

        def step(t, carry):
            bvals, bidx = carry
            c0 = c0base + t * tcol
            colx = xsT_ref[:, pl.ds(c0, tcol)]
            sqj = sq_ref[:, pl.ds(c0, tcol)]
            dots = jnp.dot(xrow, colx, preferred_element_type=jnp.float32)
            key = 2.0 * dots - sqj
            j = c0 + jax.lax.broadcasted_iota(jnp.int32, (tr, tcol), 1)
            mask = (j >= rs) & (j < re)
            key = jnp.where(mask, key, neg)
            cat_v = jnp.concatenate([bvals, key], axis=1)
            cat_i = jnp.concatenate([bidx, j], axis=1)
            pos = jax.lax.broadcasted_iota(jnp.int32, (tr, width), 1)
            nv, ni = [], []
            for _ in range(k):
                v = jnp.max(cat_v, axis=1, keepdims=True)
                elig = cat_v == v
                p = jnp.min(jnp.where(elig, pos, width), axis=1,
                            keepdims=True)
                sel = pos == p
                ik = jnp.sum(jnp.where(sel, cat_i, 0), axis=1, keepdims=True)
                nv.append(v)
                ni.append(ik)
                cat_v = jnp.where(sel, neg, cat_v)
            bvals = jnp.concatenate(
                nv + [jnp.full((tr, 128 - k), neg, jnp.float32)], axis=1)
            bidx = jnp.concatenate(
                ni + [jnp.zeros((tr, 128 - k), jnp.int32)], axis=1)
            return bvals, bidx

        _, bidx = jax.lax.fori_loop(0, nsteps, step, (bvals0, bidx0))
        seg = re - rs
        kk = jax.lax.broadcasted_iota(jnp.int32, (tr, kpad), 1)
        cand = kk - seg
        fill = jnp.where(cand < rs, cand, cand + seg)
        out_ref[...] = jnp.where(kk >= seg, fill, bidx[:, :kpad])

    npad = ((n + tcol - 1) // tcol) * tcol

    def knn(xs, row_start, row_end, tile_lo, tile_hi):
        xsT = jnp.pad(xs.T, ((0, 0), (0, npad - n)))
        sq2 = jnp.pad(jnp.sum(xs * xs, axis=1)[None, :], ((0, 0), (0, npad - n)))
        grid_spec = pltpu.PrefetchScalarGridSpec(
            num_scalar_prefetch=2,
            grid=(nt,),
            in_specs=[
                pl.BlockSpec((tr, xs.shape[1]), lambda i, *_: (i, 0)),
                pl.BlockSpec((xs.shape[1], npad), lambda i, *_: (0, 0)),
                pl.BlockSpec((1, npad), lambda i, *_: (0, 0)),
                pl.BlockSpec((tr, 1), lambda i, *_: (i, 0)),
                pl.BlockSpec((tr, 1), lambda i, *_: (i, 0)),
            ],
            out_specs=pl.BlockSpec((tr, kpad), lambda i, *_: (i, 0)),
        )
        return pl.pallas_call(
            body,
            grid_spec=grid_spec,
            interpret=interpret,
            out_shape=jax.ShapeDtypeStruct((n, kpad), jnp.int32),
        )(tile_lo, tile_hi, xs, xsT, sq2,
          row_start[:, None], row_end[:, None])

    return knn


def _lin1_pool_body(x1_ref, x2_ref, wa_ref, wb_ref, b_ref, batch_ref,
                    stats_ref, gmax_ref, gmin_ref, *, nb):
    i = pl.program_id(0)
    y = jnp.dot(x1_ref[...], wa_ref[...], preferred_element_type=jnp.float32)
    y = y + jnp.dot(x2_ref[...], wb_ref[...],
                    preferred_element_type=jnp.float32)
    y = jnp.maximum(y + b_ref[0, :][None, :], 0.0)
    s1 = jnp.sum(y, axis=0)
    s2 = jnp.sum(y * y, axis=0)
    zero_pad = jnp.zeros((6, y.shape[1]), jnp.float32)

    @pl.when(i == 0)
    def _init():
        stats_ref[...] = jnp.zeros_like(stats_ref)
        gmax_ref[...] = jnp.full_like(gmax_ref, -jnp.inf)
        gmin_ref[...] = jnp.full_like(gmin_ref, jnp.inf)

    stats_ref[...] = stats_ref[...] + jnp.concatenate(
        [s1[None, :], s2[None, :], zero_pad], axis=0)
    bvec = batch_ref[...]
    rows_max = []
    rows_min = []
    for g in range(nb):
        m = bvec == g
        rows_max.append(jnp.max(jnp.where(m, y, -jnp.inf), axis=0))
        rows_min.append(jnp.min(jnp.where(m, y, jnp.inf), axis=0))
    gmax_ref[...] = jnp.maximum(gmax_ref[...], jnp.stack(rows_max, axis=0))
    gmin_ref[...] = jnp.minimum(gmin_ref[...], jnp.stack(rows_min, axis=0))


def _lin1_pool(x1, x2, Wa_t, Wb_t, b, batch2d, tile, nb):
    n = x1.shape[0]
    f = Wa_t.shape[1]
    stats, gmax, gmin = pl.pallas_call(
        functools.partial(_lin1_pool_body, nb=nb),
        grid=(n // tile,),
        in_specs=[
            pl.BlockSpec((tile, x1.shape[1]), lambda i: (i, 0)),
            pl.BlockSpec((tile, x2.shape[1]), lambda i: (i, 0)),
            pl.BlockSpec((x1.shape[1], f), lambda i: (0, 0)),
            pl.BlockSpec((x2.shape[1], f), lambda i: (0, 0)),
            pl.BlockSpec((1, f), lambda i: (0, 0)),
            pl.BlockSpec((tile, 1), lambda i: (i, 0)),
        ],
        out_specs=[
            pl.BlockSpec((8, f), lambda i: (0, 0)),
            pl.BlockSpec((nb, f), lambda i: (0, 0)),
            pl.BlockSpec((nb, f), lambda i: (0, 0)),
        ],
        out_shape=[
            jax.ShapeDtypeStruct((8, f), jnp.float32),
            jax.ShapeDtypeStruct((nb, f), jnp.float32),
            jax.ShapeDtypeStruct((nb, f), jnp.float32),
        ],
    )(x1, x2, Wa_t, Wb_t, b.reshape(1, f), batch2d)
    return stats, gmax, gmin


def _bn_affine(stats, count, g, be):
    mean = stats[0] / count
    var = stats[1] / count - mean * mean
    inv = g / jnp.sqrt(var + _EPS)
    return inv, be - mean * inv


def kernel(pos, edge_index, batch, c1_W0, c1_b0, c1_g0, c1_be0, c1_W1, c1_b1,
           c1_g1, c1_be1, c1_W2, c1_b2, c1_g2, c1_be2, c2_W0, c2_b0, c2_g0,
           c2_be0, l1_W, l1_b, l1_g, l1_be, m_W0, m_b0, m_g0, m_be0, m_W1,
           m_b1, m_g1, m_be1, m_W2, m_b2):
    src = edge_index[0]
    dst = edge_index[1]

    xi = jnp.take(pos, dst, axis=0, mode='clip')
    xj = jnp.take(pos, src, axis=0, mode='clip')
    m0 = jnp.concatenate([xi, xj - xi, jnp.zeros((E, 2), jnp.float32)], axis=1)
    W0t = jnp.concatenate([c1_W0.T, jnp.zeros((2, 64), jnp.float32)], axis=0)

    y1, st1 = _mlp_pass(m0, W0t, c1_b0, tile=4000)
    a1, c1 = _bn_affine(st1, float(E), c1_g0, c1_be0)
    W1t = (c1_W1 * a1[None, :]).T
    b1f = c1_W1 @ c1 + c1_b1
    y2, st2 = _mlp_pass(y1, W1t, b1f, tile=4000)
    a2, c2 = _bn_affine(st2, float(E), c1_g1, c1_be1)
    W2t = (c1_W2 * a2[None, :]).T
    b2f = c1_W2 @ c2 + c1_b2
    y3, st3 = _mlp_pass(y2, W2t, b2f, tile=4000)
    a3, c3 = _bn_affine(st3, float(E), c1_g2, c1_be2)

    ymax = jax.ops.segment_max(y3, dst, num_segments=N)
    empty = jnp.isneginf(ymax[:, :1])
    x1 = jnp.where(empty, 0.0, a3[None, :] * ymax + c3[None, :])

    TR, TCOL = 400, 512
    bs = jnp.arange(B, dtype=batch.dtype)
    seg_lo = jnp.searchsorted(batch, bs, side='left').astype(jnp.int32)
    seg_hi = jnp.searchsorted(batch, bs, side='right').astype(jnp.int32)
    row_start = seg_lo[batch]
    row_end = seg_hi[batch]
    tile_lo = row_start[::TR]
    tile_hi = row_end[TR - 1::TR]
    idx = _make_knn(N, TR, TCOL, 8, K)(x1, row_start, row_end,
                                       tile_lo, tile_hi)[:, :K]
    xj2 = jnp.take(x1, idx.reshape(-1), axis=0, mode='clip').reshape(N, K, 64)
    xi2 = jnp.broadcast_to(x1[:, None, :], (N, K, 64))
    m2 = jnp.concatenate([xi2, xj2 - xi2], axis=-1).reshape(N * K, 128)
    y4, st4 = _mlp_pass(m2, c2_W0.T, c2_b0, tile=4000)
    a4, c4 = _bn_affine(st4, float(N * K), c2_g0, c2_be0)
    z4 = a4[None, :] * y4 + c4[None, :]
    x2 = jnp.max(z4.reshape(N, K, 128), axis=1)

    Wa_t = l1_W[:, :64].T
    Wb_t = l1_W[:, 64:].T
    st5, gmax, gmin = _lin1_pool(x1, x2, Wa_t, Wb_t, l1_b,
                                 batch[:, None].astype(jnp.int32),
                                 tile=1000, nb=B)
    a5, c5 = _bn_affine(st5, float(N), l1_g, l1_be)
    g = jnp.where(a5[None, :] >= 0, a5[None, :] * gmax,
                  a5[None, :] * gmin) + c5[None, :]

    def head_layer(x, W, b, gg, bb):
        y = jax.nn.relu(x @ W.T + b)
        mean = jnp.mean(y, axis=0)
        var = jnp.var(y, axis=0)
        return gg * (y - mean) / jnp.sqrt(var + _EPS) + bb

    g = head_layer(g, m_W0, m_b0, m_g0, m_be0)
    g = head_layer(g, m_W1, m_b1, m_g1, m_be1)
    return g @ m_W2.T + m_b2


# speedup vs baseline: 1.4881x; 1.4797x over previous
import functools

import jax
import jax.numpy as jnp
from jax.experimental import pallas as pl
from jax.experimental.pallas import tpu as pltpu
from jax.experimental.pallas import tpu_sc as plsc

N = 10000
E = 320000
B = 16
K = 5
NC = 40

_EPS = 1e-5


def _mlp_stats_body(x_ref, wt_ref, b_ref, y_ref, stats_ref):
    i = pl.program_id(0)
    x = x_ref[...]
    y = jnp.dot(x, wt_ref[...], preferred_element_type=jnp.float32)
    y = jnp.maximum(y + b_ref[0, :][None, :], 0.0)
    y_ref[...] = y
    s1 = jnp.sum(y, axis=0)
    s2 = jnp.sum(y * y, axis=0)

    @pl.when(i == 0)
    def _init():
        stats_ref[...] = jnp.zeros_like(stats_ref)

    upd = jnp.concatenate(
        [s1[None, :], s2[None, :], jnp.zeros((6, s1.shape[0]), jnp.float32)],
        axis=0)
    stats_ref[...] = stats_ref[...] + upd


def _mlp_pass(x, Wt, b, tile):
    M, IN = x.shape
    OUT = Wt.shape[1]
    grid = (M // tile,)
    y, stats = pl.pallas_call(
        _mlp_stats_body,
        grid=grid,
        in_specs=[
            pl.BlockSpec((tile, IN), lambda i: (i, 0)),
            pl.BlockSpec((IN, OUT), lambda i: (0, 0)),
            pl.BlockSpec((1, OUT), lambda i: (0, 0)),
        ],
        out_specs=[
            pl.BlockSpec((tile, OUT), lambda i: (i, 0)),
            pl.BlockSpec((8, OUT), lambda i: (0, 0)),
        ],
        out_shape=[
            jax.ShapeDtypeStruct((M, OUT), jnp.float32),
            jax.ShapeDtypeStruct((8, OUT), jnp.float32),
        ],
    )(x, Wt, b.reshape(1, OUT))
    return y, stats


try:
    _SC_INFO = plsc.get_sparse_core_info()
    _SC_NC = _SC_INFO.num_cores
    _SC_NW = _SC_INFO.num_cores * _SC_INFO.num_subcores
except Exception:
    _SC_NC = 2
    _SC_NW = 32


def _make_sc_gather(n_pairs, feat, out_cols, e_total, ch=128):
    per_w = e_total // _SC_NW
    assert per_w * _SC_NW == e_total and per_w % 8 == 0
    nch = per_w // ch
    tail = per_w - nch * ch
    mesh = plsc.VectorSubcoreMesh(core_axis_name="c", subcore_axis_name="s")

    del out_cols
    out_type = [jax.ShapeDtypeStruct((e_total, feat), jnp.float32)
                for _ in range(n_pairs)]
    scratch = []
    for _ in range(n_pairs):
        scratch.append(pltpu.VMEM((ch,), jnp.int32))
        scratch.append(pltpu.VMEM((ch, feat), jnp.float32))
    if tail:
        for _ in range(n_pairs):
            scratch.append(pltpu.VMEM((tail,), jnp.int32))
            scratch.append(pltpu.VMEM((tail, feat), jnp.float32))
    scratch.append(pltpu.SemaphoreType.DMA)

    @functools.partial(pl.kernel, mesh=mesh, out_type=out_type,
                       scratch_types=scratch)
    def gather_k(*refs):
        tables = refs[:n_pairs]
        idxs = refs[n_pairs:2 * n_pairs]
        outs = refs[2 * n_pairs:3 * n_pairs]
        rest = refs[3 * n_pairs:]
        sem = rest[-1]
        bufs = rest[:2 * n_pairs]
        tbufs = rest[2 * n_pairs:4 * n_pairs] if tail else None
        wid = jax.lax.axis_index("s") * _SC_NC + jax.lax.axis_index("c")
        base = wid * per_w

        def chunk(off, size, iv, rv):
            for p in range(n_pairs):
                pltpu.sync_copy(idxs[p].at[pl.ds(base + off, size)], iv[p])
                pltpu.async_copy(tables[p].at[iv[p]], rv[p], sem).wait()
                pltpu.sync_copy(rv[p], outs[p].at[pl.ds(base + off, size)])

        def body(t, _):
            chunk(t * ch, ch, [bufs[2 * p] for p in range(n_pairs)],
                  [bufs[2 * p + 1] for p in range(n_pairs)])
            return 0

        jax.lax.fori_loop(0, nch, body, 0)
        if tail:
            chunk(nch * ch, tail, [tbufs[2 * p] for p in range(n_pairs)],
                  [tbufs[2 * p + 1] for p in range(n_pairs)])

    return gather_k


def _make_knn(n, tr, tcol, kpad, k, interpret=False):
    nt = n // tr

    def body(tl_ref, th_ref, xrow_ref, xsT_ref, sq_ref, rs_ref, re_ref,
             out_ref):
        i = pl.program_id(0)
        lo = tl_ref[i]
        hi = th_ref[i]
        c0base = (lo // tcol) * tcol
        nsteps = (hi - c0base + tcol - 1) // tcol
        xrow = xrow_ref[...]
        rs = rs_ref[...]
        re = re_ref[...]
        neg = jnp.float32(-jnp.inf)
        bvals0 = jnp.full((tr, 128), neg, jnp.float32)
        bidx0 = jnp.zeros((tr, 128), jnp.int32)
        width = 128 + tcol

        def step(t, carry):
            bvals, bidx = carry
            c0 = c0base + t * tcol
            colx = xsT_ref[:, pl.ds(c0, tcol)]
            sqj = sq_ref[:, pl.ds(c0, tcol)]
            dots = jnp.dot(xrow, colx, preferred_element_type=jnp.float32)
            key = 2.0 * dots - sqj
            j = c0 + jax.lax.broadcasted_iota(jnp.int32, (tr, tcol), 1)
            mask = (j >= rs) & (j < re)
            key = jnp.where(mask, key, neg)
            cat_v = jnp.concatenate([bvals, key], axis=1)
            cat_i = jnp.concatenate([bidx, j], axis=1)
            pos = jax.lax.broadcasted_iota(jnp.int32, (tr, width), 1)
            nv, ni = [], []
            for _ in range(k):
                v = jnp.max(cat_v, axis=1, keepdims=True)
                elig = cat_v == v
                p = jnp.min(jnp.where(elig, pos, width), axis=1,
                            keepdims=True)
                sel = pos == p
                ik = jnp.sum(jnp.where(sel, cat_i, 0), axis=1, keepdims=True)
                nv.append(v)
                ni.append(ik)
                cat_v = jnp.where(sel, neg, cat_v)
            bvals = jnp.concatenate(
                nv + [jnp.full((tr, 128 - k), neg, jnp.float32)], axis=1)
            bidx = jnp.concatenate(
                ni + [jnp.zeros((tr, 128 - k), jnp.int32)], axis=1)
            return bvals, bidx

        _, bidx = jax.lax.fori_loop(0, nsteps, step, (bvals0, bidx0))
        seg = re - rs
        kk = jax.lax.broadcasted_iota(jnp.int32, (tr, kpad), 1)
        cand = kk - seg
        fill = jnp.where(cand < rs, cand, cand + seg)
        out_ref[...] = jnp.where(kk >= seg, fill, bidx[:, :kpad])

    npad = ((n + tcol - 1) // tcol) * tcol

    def knn(xs, row_start, row_end, tile_lo, tile_hi):
        xsT = jnp.pad(xs.T, ((0, 0), (0, npad - n)))
        sq2 = jnp.pad(jnp.sum(xs * xs, axis=1)[None, :], ((0, 0), (0, npad - n)))
        grid_spec = pltpu.PrefetchScalarGridSpec(
            num_scalar_prefetch=2,
            grid=(nt,),
            in_specs=[
                pl.BlockSpec((tr, xs.shape[1]), lambda i, *_: (i, 0)),
                pl.BlockSpec((xs.shape[1], npad), lambda i, *_: (0, 0)),
                pl.BlockSpec((1, npad), lambda i, *_: (0, 0)),
                pl.BlockSpec((tr, 1), lambda i, *_: (i, 0)),
                pl.BlockSpec((tr, 1), lambda i, *_: (i, 0)),
            ],
            out_specs=pl.BlockSpec((tr, kpad), lambda i, *_: (i, 0)),
        )
        return pl.pallas_call(
            body,
            grid_spec=grid_spec,
            interpret=interpret,
            out_shape=jax.ShapeDtypeStruct((n, kpad), jnp.int32),
        )(tile_lo, tile_hi, xs, xsT, sq2,
          row_start[:, None], row_end[:, None])

    return knn


def _lin1_pool_body(x1_ref, x2_ref, wa_ref, wb_ref, b_ref, batch_ref,
                    stats_ref, gmax_ref, gmin_ref, *, nb):
    i = pl.program_id(0)
    y = jnp.dot(x1_ref[...], wa_ref[...], preferred_element_type=jnp.float32)
    y = y + jnp.dot(x2_ref[...], wb_ref[...],
                    preferred_element_type=jnp.float32)
    y = jnp.maximum(y + b_ref[0, :][None, :], 0.0)
    s1 = jnp.sum(y, axis=0)
    s2 = jnp.sum(y * y, axis=0)
    zero_pad = jnp.zeros((6, y.shape[1]), jnp.float32)

    @pl.when(i == 0)
    def _init():
        stats_ref[...] = jnp.zeros_like(stats_ref)
        gmax_ref[...] = jnp.full_like(gmax_ref, -jnp.inf)
        gmin_ref[...] = jnp.full_like(gmin_ref, jnp.inf)

    stats_ref[...] = stats_ref[...] + jnp.concatenate(
        [s1[None, :], s2[None, :], zero_pad], axis=0)
    bvec = batch_ref[...]
    rows_max = []
    rows_min = []
    for g in range(nb):
        m = bvec == g
        rows_max.append(jnp.max(jnp.where(m, y, -jnp.inf), axis=0))
        rows_min.append(jnp.min(jnp.where(m, y, jnp.inf), axis=0))
    gmax_ref[...] = jnp.maximum(gmax_ref[...], jnp.stack(rows_max, axis=0))
    gmin_ref[...] = jnp.minimum(gmin_ref[...], jnp.stack(rows_min, axis=0))


def _lin1_pool(x1, x2, Wa_t, Wb_t, b, batch2d, tile, nb):
    n = x1.shape[0]
    f = Wa_t.shape[1]
    stats, gmax, gmin = pl.pallas_call(
        functools.partial(_lin1_pool_body, nb=nb),
        grid=(n // tile,),
        in_specs=[
            pl.BlockSpec((tile, x1.shape[1]), lambda i: (i, 0)),
            pl.BlockSpec((tile, x2.shape[1]), lambda i: (i, 0)),
            pl.BlockSpec((x1.shape[1], f), lambda i: (0, 0)),
            pl.BlockSpec((x2.shape[1], f), lambda i: (0, 0)),
            pl.BlockSpec((1, f), lambda i: (0, 0)),
            pl.BlockSpec((tile, 1), lambda i: (i, 0)),
        ],
        out_specs=[
            pl.BlockSpec((8, f), lambda i: (0, 0)),
            pl.BlockSpec((nb, f), lambda i: (0, 0)),
            pl.BlockSpec((nb, f), lambda i: (0, 0)),
        ],
        out_shape=[
            jax.ShapeDtypeStruct((8, f), jnp.float32),
            jax.ShapeDtypeStruct((nb, f), jnp.float32),
            jax.ShapeDtypeStruct((nb, f), jnp.float32),
        ],
    )(x1, x2, Wa_t, Wb_t, b.reshape(1, f), batch2d)
    return stats, gmax, gmin


def _bn_affine(stats, count, g, be):
    mean = stats[0] / count
    var = stats[1] / count - mean * mean
    inv = g / jnp.sqrt(var + _EPS)
    return inv, be - mean * inv


def kernel(pos, edge_index, batch, c1_W0, c1_b0, c1_g0, c1_be0, c1_W1, c1_b1,
           c1_g1, c1_be1, c1_W2, c1_b2, c1_g2, c1_be2, c2_W0, c2_b0, c2_g0,
           c2_be0, l1_W, l1_b, l1_g, l1_be, m_W0, m_b0, m_g0, m_be0, m_W1,
           m_b1, m_g1, m_be1, m_W2, m_b2):
    src = edge_index[0]
    dst = edge_index[1]

    pos128 = jnp.pad(pos, ((0, 0), (0, 125)))
    xi128, xj128 = _make_sc_gather(2, 128, 16, E)(pos128, pos128, dst, src)
    xi = xi128[:, :3]
    xj = xj128[:, :3]
    m0 = jnp.concatenate([xi, xj - xi, jnp.zeros((E, 2), jnp.float32)], axis=1)
    W0t = jnp.concatenate([c1_W0.T, jnp.zeros((2, 64), jnp.float32)], axis=0)

    y1, st1 = _mlp_pass(m0, W0t, c1_b0, tile=4000)
    a1, c1 = _bn_affine(st1, float(E), c1_g0, c1_be0)
    W1t = (c1_W1 * a1[None, :]).T
    b1f = c1_W1 @ c1 + c1_b1
    y2, st2 = _mlp_pass(y1, W1t, b1f, tile=4000)
    a2, c2 = _bn_affine(st2, float(E), c1_g1, c1_be1)
    W2t = (c1_W2 * a2[None, :]).T
    b2f = c1_W2 @ c2 + c1_b2
    y3, st3 = _mlp_pass(y2, W2t, b2f, tile=4000)
    a3, c3 = _bn_affine(st3, float(E), c1_g2, c1_be2)

    ymax = jax.ops.segment_max(y3, dst, num_segments=N)
    empty = jnp.isneginf(ymax[:, :1])
    x1 = jnp.where(empty, 0.0, a3[None, :] * ymax + c3[None, :])

    TR, TCOL = 400, 512
    bs = jnp.arange(B, dtype=batch.dtype)
    seg_lo = jnp.searchsorted(batch, bs, side='left').astype(jnp.int32)
    seg_hi = jnp.searchsorted(batch, bs, side='right').astype(jnp.int32)
    row_start = seg_lo[batch]
    row_end = seg_hi[batch]
    tile_lo = row_start[::TR]
    tile_hi = row_end[TR - 1::TR]
    idx = _make_knn(N, TR, TCOL, 8, K)(x1, row_start, row_end,
                                       tile_lo, tile_hi)[:, :K]
    EP = 50176
    idx_flat = jnp.concatenate(
        [idx.reshape(-1), jnp.zeros((EP - N * K,), jnp.int32)])
    x1pad = jnp.pad(x1, ((0, 0), (0, 64)))
    (xj2f,) = _make_sc_gather(1, 128, 64, EP)(x1pad, idx_flat)
    xj2 = xj2f[:N * K, :64].reshape(N, K, 64)
    xi2 = jnp.broadcast_to(x1[:, None, :], (N, K, 64))
    m2 = jnp.concatenate([xi2, xj2 - xi2], axis=-1).reshape(N * K, 128)
    y4, st4 = _mlp_pass(m2, c2_W0.T, c2_b0, tile=4000)
    a4, c4 = _bn_affine(st4, float(N * K), c2_g0, c2_be0)
    z4 = a4[None, :] * y4 + c4[None, :]
    x2 = jnp.max(z4.reshape(N, K, 128), axis=1)

    Wa_t = l1_W[:, :64].T
    Wb_t = l1_W[:, 64:].T
    st5, gmax, gmin = _lin1_pool(x1, x2, Wa_t, Wb_t, l1_b,
                                 batch[:, None].astype(jnp.int32),
                                 tile=1000, nb=B)
    a5, c5 = _bn_affine(st5, float(N), l1_g, l1_be)
    g = jnp.where(a5[None, :] >= 0, a5[None, :] * gmax,
                  a5[None, :] * gmin) + c5[None, :]

    def head_layer(x, W, b, gg, bb):
        y = jax.nn.relu(x @ W.T + b)
        mean = jnp.mean(y, axis=0)
        var = jnp.var(y, axis=0)
        return gg * (y - mean) / jnp.sqrt(var + _EPS) + bb

    g = head_layer(g, m_W0, m_b0, m_g0, m_be0)
    g = head_layer(g, m_W1, m_b1, m_g1, m_be1)
    return g @ m_W2.T + m_b2
